# baseline (speedup 1.0000x reference)
import functools

import jax
import jax.numpy as jnp
from jax import lax
from jax.experimental import pallas as pl
from jax.experimental.pallas import tpu as pltpu
from jax.experimental.pallas import tpu_sc as plsc

_ROWS, _COLS = 16384, 512
_OUT_COLS = _COLS // 2
_NC, _NS, _L = 2, 16, 16
_NW = _NC * _NS
_IN_PER_W = _ROWS * _COLS // _NW
_CHUNK_IN = 32768
_CHUNK_OUT = _CHUNK_IN // 2
_NCHUNK = _IN_PER_W // _CHUNK_IN
_UNROLL = 8
_NVREG = _CHUNK_OUT // _L


def _body(x_hbm, out_hbm, in_buf, out_buf):
    wid = lax.axis_index("s") * _NC + lax.axis_index("c")
    lane = lax.iota(jnp.int32, 16)

    def chunk_body(c, _):
        in0 = wid * _IN_PER_W + c * _CHUNK_IN
        out0 = wid * (_IN_PER_W // 2) + c * _CHUNK_OUT
        pltpu.sync_copy(x_hbm.at[pl.ds(in0, _CHUNK_IN)], in_buf)

        def t_body(i, _):
            o = i * (_UNROLL * _L)
            s = i * (_UNROLL * 2 * _L)
            for u in range(_UNROLL):
                idx = s + (2 * _L * u) + 2 * lane
                v = plsc.load_gather(in_buf, [idx])
                out_buf[pl.ds(o + _L * u, _L)] = v
            return 0

        lax.fori_loop(0, _NVREG // _UNROLL, t_body, 0)
        pltpu.sync_copy(out_buf, out_hbm.at[pl.ds(out0, _CHUNK_OUT)])
        return 0

    lax.fori_loop(0, _NCHUNK, chunk_body, 0)


_deinterleave = functools.partial(
    pl.kernel,
    out_type=jax.ShapeDtypeStruct((_ROWS * _OUT_COLS,), jnp.float32),
    mesh=plsc.VectorSubcoreMesh(core_axis_name="c", subcore_axis_name="s"),
    scratch_types=[
        pltpu.VMEM((_CHUNK_IN,), jnp.float32),
        pltpu.VMEM((_CHUNK_OUT,), jnp.float32),
    ],
    compiler_params=pltpu.CompilerParams(needs_layout_passes=False),
)(_body)


def kernel(x):
    out_flat = _deinterleave(x.reshape(-1))
    return out_flat.reshape(_ROWS, _OUT_COLS)


# speedup vs baseline: 1.1416x; 1.1416x over previous
import functools

import jax
import jax.numpy as jnp
from jax import lax
from jax.experimental import pallas as pl
from jax.experimental.pallas import tpu as pltpu
from jax.experimental.pallas import tpu_sc as plsc

_ROWS, _COLS = 16384, 512
_OUT_COLS = _COLS // 2
_NC, _NS, _L = 2, 16, 16
_NW = _NC * _NS
_IN_PER_W = _ROWS * _COLS // _NW
_OUT_PER_W = _IN_PER_W // 2
_CHUNK_IN = 32768
_CHUNK_OUT = _CHUNK_IN // 2
_NCHUNK = _IN_PER_W // _CHUNK_IN
_UNROLL = 8
_NVREG = _CHUNK_OUT // _L


def _body(x_hbm, out_hbm, in0, in1, out0, out1, si0, si1, so0, so1):
    wid = lax.axis_index("s") * _NC + lax.axis_index("c")
    lanes2 = 2 * lax.iota(jnp.int32, 16)
    base_in = wid * _IN_PER_W
    base_out = wid * _OUT_PER_W
    ins, outs, sis, sos = (in0, in1), (out0, out1), (si0, si1), (so0, so1)

    in_h = [None] * _NCHUNK
    out_h = [None] * _NCHUNK
    for c in range(2):
        in_h[c] = pltpu.async_copy(
            x_hbm.at[pl.ds(base_in + c * _CHUNK_IN, _CHUNK_IN)], ins[c], sis[c]
        )
    for c in range(_NCHUNK):
        b = c % 2
        in_h[c].wait()
        if c >= 2:
            out_h[c - 2].wait()

        def t_body(i, _, ib=ins[b], ob=outs[b]):
            s = i * (_UNROLL * 2 * _L)
            o = i * (_UNROLL * _L)
            for u in range(_UNROLL):
                idx = s + (2 * _L * u) + lanes2
                v = plsc.load_gather(ib, [idx])
                ob[pl.ds(o + _L * u, _L)] = v
            return 0

        lax.fori_loop(0, _NVREG // _UNROLL, t_body, 0)
        out_h[c] = pltpu.async_copy(
            outs[b], out_hbm.at[pl.ds(base_out + c * _CHUNK_OUT, _CHUNK_OUT)], sos[b]
        )
        if c + 2 < _NCHUNK:
            in_h[c + 2] = pltpu.async_copy(
                x_hbm.at[pl.ds(base_in + (c + 2) * _CHUNK_IN, _CHUNK_IN)],
                ins[b],
                sis[b],
            )
    out_h[_NCHUNK - 2].wait()
    out_h[_NCHUNK - 1].wait()


_deinterleave = functools.partial(
    pl.kernel,
    out_type=jax.ShapeDtypeStruct((_ROWS * _OUT_COLS,), jnp.float32),
    mesh=plsc.VectorSubcoreMesh(core_axis_name="c", subcore_axis_name="s"),
    scratch_types=[
        pltpu.VMEM((_CHUNK_IN,), jnp.float32),
        pltpu.VMEM((_CHUNK_IN,), jnp.float32),
        pltpu.VMEM((_CHUNK_OUT,), jnp.float32),
        pltpu.VMEM((_CHUNK_OUT,), jnp.float32),
        pltpu.SemaphoreType.DMA,
        pltpu.SemaphoreType.DMA,
        pltpu.SemaphoreType.DMA,
        pltpu.SemaphoreType.DMA,
    ],
    compiler_params=pltpu.CompilerParams(needs_layout_passes=False),
)(_body)


def kernel(x):
    out_flat = _deinterleave(x.reshape(-1))
    return out_flat.reshape(_ROWS, _OUT_COLS)


# speedup vs baseline: 2.0559x; 1.8009x over previous
import functools

import jax
import jax.numpy as jnp
from jax import lax
from jax.experimental import pallas as pl
from jax.experimental.pallas import tpu as pltpu
from jax.experimental.pallas import tpu_sc as plsc

_ROWS, _COLS = 16384, 512
_OUT_COLS = _COLS // 2
_NC, _NS, _L = 2, 16, 16
_NW = _NC * _NS
_RPW = _ROWS // _NW
_CHUNK = 64
_NCHUNK = _RPW // _CHUNK
_JPR = _OUT_COLS // _L


def _body(x_hbm, out_hbm, in0, in1, out0, out1, si0, si1, so0, so1):
    wid = lax.axis_index("s") * _NC + lax.axis_index("c")
    lane = lax.iota(jnp.int32, 16)
    row0 = wid * _RPW
    ins, outs, sis, sos = (in0, in1), (out0, out1), (si0, si1), (so0, so1)

    in_h = [None] * _NCHUNK
    out_h = [None] * _NCHUNK
    for c in range(2):
        in_h[c] = pltpu.async_copy(
            x_hbm.at[pl.ds(row0 + c * _CHUNK, _CHUNK)], ins[c], sis[c]
        )
    for c in range(_NCHUNK):
        b = c % 2
        in_h[c].wait()
        if c >= 2:
            out_h[c - 2].wait()

        def r_body(r, _, ib=ins[b], ob=outs[b]):
            rowv = jnp.full((16,), r, jnp.int32)
            for j in range(_JPR):
                col = 2 * _L * j + 2 * lane
                v = plsc.load_gather(ib, [rowv, col])
                ob[r, pl.ds(_L * j, _L)] = v
            return 0

        lax.fori_loop(0, _CHUNK, r_body, 0)
        out_h[c] = pltpu.async_copy(
            outs[b], out_hbm.at[pl.ds(row0 + c * _CHUNK, _CHUNK)], sos[b]
        )
        if c + 2 < _NCHUNK:
            in_h[c + 2] = pltpu.async_copy(
                x_hbm.at[pl.ds(row0 + (c + 2) * _CHUNK, _CHUNK)], ins[b], sis[b]
            )
    out_h[_NCHUNK - 2].wait()
    out_h[_NCHUNK - 1].wait()


_deinterleave = functools.partial(
    pl.kernel,
    out_type=jax.ShapeDtypeStruct((_ROWS, _OUT_COLS), jnp.float32),
    mesh=plsc.VectorSubcoreMesh(core_axis_name="c", subcore_axis_name="s"),
    scratch_types=[
        pltpu.VMEM((_CHUNK, _COLS), jnp.float32),
        pltpu.VMEM((_CHUNK, _COLS), jnp.float32),
        pltpu.VMEM((_CHUNK, _OUT_COLS), jnp.float32),
        pltpu.VMEM((_CHUNK, _OUT_COLS), jnp.float32),
        pltpu.SemaphoreType.DMA,
        pltpu.SemaphoreType.DMA,
        pltpu.SemaphoreType.DMA,
        pltpu.SemaphoreType.DMA,
    ],
    compiler_params=pltpu.CompilerParams(needs_layout_passes=False),
)(_body)


def kernel(x):
    return _deinterleave(x)


# speedup vs baseline: 2.1346x; 1.0383x over previous
import functools

import jax
import jax.numpy as jnp
from jax import lax
from jax.experimental import pallas as pl
from jax.experimental.pallas import tpu as pltpu
from jax.experimental.pallas import tpu_sc as plsc

_ROWS, _COLS = 16384, 512
_OUT_COLS = _COLS // 2
_NC, _NS, _L = 2, 16, 16
_NW = _NC * _NS
_RPW = _ROWS // _NW
_CHUNK = 64
_NCHUNK = _RPW // _CHUNK
_NGROUP = _NCHUNK // 2
_JPR = _OUT_COLS // _L


def _body(x_hbm, out_hbm, in0, in1, out0, out1, si0, si1, so0, so1):
    wid = lax.axis_index("s") * _NC + lax.axis_index("c")
    lane = lax.iota(jnp.int32, 16)
    row0 = wid * _RPW
    ins, outs, sis, sos = (in0, in1), (out0, out1), (si0, si1), (so0, so1)

    for b in range(2):
        pltpu.async_copy(
            x_hbm.at[pl.ds(row0 + b * _CHUNK, _CHUNK)], ins[b], sis[b]
        )

    def group(g, _):
        for b in range(2):
            c = 2 * g + b
            r0 = row0 + c * _CHUNK
            pltpu.make_async_copy(
                x_hbm.at[pl.ds(r0, _CHUNK)], ins[b], sis[b]
            ).wait()

            @pl.when(g > 0)
            def _():
                pltpu.make_async_copy(
                    outs[b], out_hbm.at[pl.ds(r0, _CHUNK)], sos[b]
                ).wait()

            def r_body(r, _, ib=ins[b], ob=outs[b]):
                rowv = jnp.full((16,), r, jnp.int32)
                for j in range(_JPR):
                    col = 2 * _L * j + 2 * lane
                    v = plsc.load_gather(ib, [rowv, col])
                    ob[r, pl.ds(_L * j, _L)] = v
                return 0

            lax.fori_loop(0, _CHUNK, r_body, 0)
            pltpu.async_copy(
                outs[b], out_hbm.at[pl.ds(r0, _CHUNK)], sos[b]
            )

            @pl.when(g < _NGROUP - 1)
            def _():
                pltpu.async_copy(
                    x_hbm.at[pl.ds(r0 + 2 * _CHUNK, _CHUNK)], ins[b], sis[b]
                )

        return 0

    lax.fori_loop(0, _NGROUP, group, 0)
    for b in range(2):
        pltpu.make_async_copy(
            outs[b], out_hbm.at[pl.ds(row0, _CHUNK)], sos[b]
        ).wait()


_deinterleave = functools.partial(
    pl.kernel,
    out_type=jax.ShapeDtypeStruct((_ROWS, _OUT_COLS), jnp.float32),
    mesh=plsc.VectorSubcoreMesh(core_axis_name="c", subcore_axis_name="s"),
    scratch_types=[
        pltpu.VMEM((_CHUNK, _COLS), jnp.float32),
        pltpu.VMEM((_CHUNK, _COLS), jnp.float32),
        pltpu.VMEM((_CHUNK, _OUT_COLS), jnp.float32),
        pltpu.VMEM((_CHUNK, _OUT_COLS), jnp.float32),
        pltpu.SemaphoreType.DMA,
        pltpu.SemaphoreType.DMA,
        pltpu.SemaphoreType.DMA,
        pltpu.SemaphoreType.DMA,
    ],
    compiler_params=pltpu.CompilerParams(needs_layout_passes=False),
)(_body)


def kernel(x):
    return _deinterleave(x)
